# Initial kernel scaffold; baseline (speedup 1.0000x reference)
import jax
import jax.numpy as jnp
from jax.experimental import pallas as pl


def kernel(r, xyz, t0_W1, t0_b1, t0_W2, t0_b2, k_W1, k_b1, k_W2, k_b2, angles, num_angles):
    raise NotImplementedError("write your pallas kernel here")


# speedup vs baseline: 5.2461x; 5.2461x over previous
import dataclasses
import functools

import jax
import jax.numpy as jnp
import numpy as np
from jax import lax
from jax.experimental import pallas as pl
from jax.experimental.pallas import tpu as pltpu
from jax.experimental.pallas import tpu_sc as plsc

N_NODES = 8192
FR = 128
LH = 256
N_ANGLES = 130816
N_MOL = 512
NP = 131072
GW = 64
TB = 2048
XW = 16


NW = 32
PW = NP // NW
CH = 64


def _sc_gather(r, xyzt, i0, i1, i2):
  mesh = plsc.VectorSubcoreMesh(core_axis_name="c", subcore_axis_name="s")
  f32 = jnp.float32
  out_type = (
      jax.ShapeDtypeStruct((NP, FR), f32),
      jax.ShapeDtypeStruct((NP, FR), f32),
      jax.ShapeDtypeStruct((NP, FR), f32),
      jax.ShapeDtypeStruct((NP,), f32),
      jax.ShapeDtypeStruct((NP,), f32),
      jax.ShapeDtypeStruct((NP,), f32),
  )
  scratch_types = [
      pltpu.VMEM((PW,), jnp.int32),
      pltpu.VMEM((PW,), jnp.int32),
      pltpu.VMEM((PW,), jnp.int32),
      pltpu.VMEM((N_NODES,), f32),
      pltpu.VMEM((N_NODES,), f32),
      pltpu.VMEM((N_NODES,), f32),
      pltpu.VMEM((CH, FR), f32),
      pltpu.VMEM((CH, FR), f32),
      pltpu.VMEM((CH, FR), f32),
      pltpu.VMEM((PW,), f32),
      pltpu.VMEM((PW,), f32),
      pltpu.VMEM((PW,), f32),
      pltpu.SemaphoreType.DMA,
  ]

  cp = pltpu.CompilerParams()
  if "needs_layout_passes" in pltpu.CompilerParams.__dataclass_fields__:
    cp = dataclasses.replace(cp, needs_layout_passes=False)

  @functools.partial(pl.kernel, out_type=out_type, mesh=mesh,
                     scratch_types=scratch_types, compiler_params=cp)
  def k(r_hbm, x_hbm, i0_hbm, i1_hbm, i2_hbm,
        g0, g1, g2, dh, n1h, n2h,
        iv0, iv1, iv2, xs, ys, zs, b0, b1, b2, dv, n1v, n2v, sem):
    wid = lax.axis_index("s") * 2 + lax.axis_index("c")
    base = wid * PW
    pltpu.sync_copy(x_hbm.at[pl.ds(0, N_NODES)], xs)
    pltpu.sync_copy(x_hbm.at[pl.ds(N_NODES, N_NODES)], ys)
    pltpu.sync_copy(x_hbm.at[pl.ds(2 * N_NODES, N_NODES)], zs)
    pltpu.sync_copy(i0_hbm.at[pl.ds(base, PW)], iv0)
    pltpu.sync_copy(i1_hbm.at[pl.ds(base, PW)], iv1)
    pltpu.sync_copy(i2_hbm.at[pl.ds(base, PW)], iv2)

    @pl.loop(0, PW // CH)
    def _(c):
      off = c * CH
      cp0 = pltpu.async_copy(r_hbm.at[iv0.at[pl.ds(off, CH)]], b0, sem)
      cp1 = pltpu.async_copy(r_hbm.at[iv1.at[pl.ds(off, CH)]], b1, sem)
      cp2 = pltpu.async_copy(r_hbm.at[iv2.at[pl.ds(off, CH)]], b2, sem)
      for kk in range(CH // 16):
        o = off + 16 * kk
        j0 = iv0[pl.ds(o, 16)]
        j1 = iv1[pl.ds(o, 16)]
        j2 = iv2[pl.ds(o, 16)]
        x0 = plsc.load_gather(xs, [j0])
        x1 = plsc.load_gather(xs, [j1])
        x2 = plsc.load_gather(xs, [j2])
        y0 = plsc.load_gather(ys, [j0])
        y1 = plsc.load_gather(ys, [j1])
        y2 = plsc.load_gather(ys, [j2])
        z0 = plsc.load_gather(zs, [j0])
        z1 = plsc.load_gather(zs, [j1])
        z2 = plsc.load_gather(zs, [j2])
        d1x = x1 - x0
        d1y = y1 - y0
        d1z = z1 - z0
        d2x = x2 - x1
        d2y = y2 - y1
        d2z = z2 - z1
        dv[pl.ds(o, 16)] = -(d1x * d2x + d1y * d2y + d1z * d2z)
        n1v[pl.ds(o, 16)] = d1x * d1x + d1y * d1y + d1z * d1z
        n2v[pl.ds(o, 16)] = d2x * d2x + d2y * d2y + d2z * d2z
      cp0.wait()
      cp1.wait()
      cp2.wait()
      pltpu.sync_copy(b0, g0.at[pl.ds(base + off, CH)])
      pltpu.sync_copy(b1, g1.at[pl.ds(base + off, CH)])
      pltpu.sync_copy(b2, g2.at[pl.ds(base + off, CH)])

    pltpu.sync_copy(dv, dh.at[pl.ds(base, PW)])
    pltpu.sync_copy(n1v, n1h.at[pl.ds(base, PW)])
    pltpu.sync_copy(n2v, n2h.at[pl.ds(base, PW)])

  return k(r, xyzt, i0, i1, i2)


def _arccos(x):
  ax = jnp.abs(x)
  p = jnp.float32(-0.0012624911)
  p = p * ax + jnp.float32(0.0066700901)
  p = p * ax + jnp.float32(-0.0170881256)
  p = p * ax + jnp.float32(0.0308918810)
  p = p * ax + jnp.float32(-0.0501743046)
  p = p * ax + jnp.float32(0.0889789874)
  p = p * ax + jnp.float32(-0.2145988016)
  p = p * ax + jnp.float32(1.5707963050)
  r = jnp.sqrt(jnp.maximum(1.0 - ax, 0.0)) * p
  return jnp.where(x >= 0, r, jnp.float32(np.pi) - r)


def _tc_body(g0_r, g1_r, g2_r, d_r, n1_r, n2_r, w1_r, b1_r, w2_r,
             lo_r, hi_r, c_r, out_r):
  i = pl.program_id(0)
  x1 = g0_r[...] + g2_r[...]
  x2 = g1_r[...]
  w1 = w1_r[...]
  h = jnp.dot(x1, w1[:FR, :], preferred_element_type=jnp.float32)
  h = h + jnp.dot(x2, w1[FR:, :], preferred_element_type=jnp.float32)
  h = jnp.tanh(h + b1_r[...])
  m = h * w2_r[...]
  t0m = jnp.sum(m[:, :LH], axis=1, keepdims=True)
  km = jnp.sum(m[:, LH:], axis=1, keepdims=True)

  dot = d_r[...]
  n1 = n1_r[...]
  n2 = n2_r[...]
  cos = dot * lax.rsqrt(n1 * n2)
  theta_l = _arccos(cos * jnp.float32(1.0 / 1.000001))
  theta = jnp.transpose(theta_l)

  t0h = (c_r[0] + t0m) ** 2
  kh = (c_r[1] + km) ** 2
  e = 0.5 * kh * (theta - t0h) ** 2

  rows = lax.broadcasted_iota(jnp.int32, (TB, N_MOL), 0) + i * TB
  oh = (rows >= lo_r[...]) & (rows < hi_r[...])
  contrib = jnp.where(oh, e, 0.0)
  part = jnp.sum(contrib, axis=0, keepdims=True)

  @pl.when(i == 0)
  def _():
    out_r[...] = jnp.zeros_like(out_r)

  out_r[...] += part


def _tc_compute(g0, g1, g2, d, n1, n2, w1c, b1c, w2r, lo, hi, consts):
  grid = (NP // TB,)
  full = lambda i: (0, 0)
  blk = lambda i: (i, 0)
  lane = lambda i: (0, i)
  return pl.pallas_call(
      _tc_body,
      grid=grid,
      in_specs=[pl.BlockSpec((TB, FR), blk)] * 3
      + [pl.BlockSpec((1, TB), lane)] * 3
      + [
          pl.BlockSpec((2 * FR, 2 * LH), full),
          pl.BlockSpec((1, 2 * LH), full),
          pl.BlockSpec((1, 2 * LH), full),
          pl.BlockSpec((1, N_MOL), full),
          pl.BlockSpec((1, N_MOL), full),
          pl.BlockSpec(memory_space=pltpu.SMEM),
      ],
      out_specs=pl.BlockSpec((1, N_MOL), full),
      out_shape=jax.ShapeDtypeStruct((1, N_MOL), jnp.float32),
  )(g0, g1, g2, d, n1, n2, w1c, b1c, w2r, lo, hi, consts)


def kernel(r, xyz, t0_W1, t0_b1, t0_W2, t0_b2, k_W1, k_b1, k_W2, k_b2,
           angles, num_angles):
  f32 = jnp.float32
  angles = angles.astype(jnp.int32)
  pad_tri = jnp.array([[0, 1, 2]], dtype=jnp.int32)
  pad = jnp.broadcast_to(pad_tri, (NP - N_ANGLES, 3))
  ap = jnp.concatenate([angles, pad], axis=0)
  i0 = ap[:, 0]
  i1 = ap[:, 1]
  i2 = ap[:, 2]

  xyzt = xyz.astype(f32).T.reshape(3 * N_NODES)

  g0, g1, g2, d, n1, n2 = _sc_gather(r.astype(f32), xyzt, i0, i1, i2)
  d = d.reshape(1, NP)
  n1 = n1.reshape(1, NP)
  n2 = n2.reshape(1, NP)

  w1c = jnp.concatenate([t0_W1, k_W1], axis=1)
  b1c = jnp.concatenate([t0_b1, k_b1]).reshape(1, 2 * LH)
  w2r = jnp.concatenate([t0_W2[:, 0], k_W2[:, 0]]).reshape(1, 2 * LH)

  ends = jnp.cumsum(num_angles.astype(jnp.int32))
  lo = (ends - num_angles.astype(jnp.int32)).reshape(1, N_MOL)
  hi = ends.reshape(1, N_MOL)

  c0 = np.float32((109.5 * np.pi / 180.0) ** 0.5)
  c1 = np.float32(10.0 ** 0.5)
  consts = jnp.stack([c0 + t0_b2[0], c1 + k_b2[0]]).astype(f32)

  out = _tc_compute(g0, g1, g2, d, n1, n2, w1c, b1c, w2r, lo, hi, consts)
  return out.reshape(N_MOL, 1)


# speedup vs baseline: 6.7584x; 1.2883x over previous
import dataclasses
import functools

import jax
import jax.numpy as jnp
import numpy as np
from jax import lax
from jax.experimental import pallas as pl
from jax.experimental.pallas import tpu as pltpu
from jax.experimental.pallas import tpu_sc as plsc

N_NODES = 8192
FR = 128
LH = 256
N_ANGLES = 130816
N_MOL = 512
NP = 131072
NSLICE = 4
SL = NP // NSLICE
NW = 32
GPW = NP // NW
SPW = SL // NW
CH = 128
NCH = SPW // CH
TB = 2048


def _sc_compiler_params():
  cp = pltpu.CompilerParams()
  if "needs_layout_passes" in pltpu.CompilerParams.__dataclass_fields__:
    cp = dataclasses.replace(cp, needs_layout_passes=False)
  return cp


def _sc_geometry(xyzt, i0, i1, i2):
  mesh = plsc.VectorSubcoreMesh(core_axis_name="c", subcore_axis_name="s")
  f32 = jnp.float32
  out_type = (
      jax.ShapeDtypeStruct((NP,), f32),
      jax.ShapeDtypeStruct((NP,), f32),
      jax.ShapeDtypeStruct((NP,), f32),
  )
  scratch_types = [
      pltpu.VMEM((GPW,), jnp.int32),
      pltpu.VMEM((GPW,), jnp.int32),
      pltpu.VMEM((GPW,), jnp.int32),
      pltpu.VMEM((N_NODES,), f32),
      pltpu.VMEM((N_NODES,), f32),
      pltpu.VMEM((N_NODES,), f32),
      pltpu.VMEM((GPW,), f32),
      pltpu.VMEM((GPW,), f32),
      pltpu.VMEM((GPW,), f32),
  ]

  @functools.partial(pl.kernel, out_type=out_type, mesh=mesh,
                     scratch_types=scratch_types,
                     compiler_params=_sc_compiler_params())
  def k(x_hbm, i0_hbm, i1_hbm, i2_hbm, dh, n1h, n2h,
        iv0, iv1, iv2, xs, ys, zs, dv, n1v, n2v):
    wid = lax.axis_index("s") * 2 + lax.axis_index("c")
    base = wid * GPW
    pltpu.sync_copy(x_hbm.at[pl.ds(0, N_NODES)], xs)
    pltpu.sync_copy(x_hbm.at[pl.ds(N_NODES, N_NODES)], ys)
    pltpu.sync_copy(x_hbm.at[pl.ds(2 * N_NODES, N_NODES)], zs)
    pltpu.sync_copy(i0_hbm.at[pl.ds(base, GPW)], iv0)
    pltpu.sync_copy(i1_hbm.at[pl.ds(base, GPW)], iv1)
    pltpu.sync_copy(i2_hbm.at[pl.ds(base, GPW)], iv2)

    @pl.loop(0, GPW // 16)
    def _(c):
      o = c * 16
      j0 = iv0[pl.ds(o, 16)]
      j1 = iv1[pl.ds(o, 16)]
      j2 = iv2[pl.ds(o, 16)]
      x0 = plsc.load_gather(xs, [j0])
      x1 = plsc.load_gather(xs, [j1])
      x2 = plsc.load_gather(xs, [j2])
      y0 = plsc.load_gather(ys, [j0])
      y1 = plsc.load_gather(ys, [j1])
      y2 = plsc.load_gather(ys, [j2])
      z0 = plsc.load_gather(zs, [j0])
      z1 = plsc.load_gather(zs, [j1])
      z2 = plsc.load_gather(zs, [j2])
      d1x = x1 - x0
      d1y = y1 - y0
      d1z = z1 - z0
      d2x = x2 - x1
      d2y = y2 - y1
      d2z = z2 - z1
      dv[pl.ds(o, 16)] = -(d1x * d2x + d1y * d2y + d1z * d2z)
      n1v[pl.ds(o, 16)] = d1x * d1x + d1y * d1y + d1z * d1z
      n2v[pl.ds(o, 16)] = d2x * d2x + d2y * d2y + d2z * d2z

    pltpu.sync_copy(dv, dh.at[pl.ds(base, GPW)])
    pltpu.sync_copy(n1v, n1h.at[pl.ds(base, GPW)])
    pltpu.sync_copy(n2v, n2h.at[pl.ds(base, GPW)])

  return k(xyzt, i0, i1, i2)


def _sc_gather_slice(r, i0, i1, i2):
  mesh = plsc.VectorSubcoreMesh(core_axis_name="c", subcore_axis_name="s")
  f32 = jnp.float32
  out_type = (
      jax.ShapeDtypeStruct((SL, FR), f32),
      jax.ShapeDtypeStruct((SL, FR), f32),
      jax.ShapeDtypeStruct((SL, FR), f32),
  )
  scratch_types = [
      pltpu.VMEM((SPW,), jnp.int32),
      pltpu.VMEM((SPW,), jnp.int32),
      pltpu.VMEM((SPW,), jnp.int32),
      pltpu.VMEM((2, CH, FR), f32),
      pltpu.VMEM((2, CH, FR), f32),
      pltpu.VMEM((2, CH, FR), f32),
      pltpu.SemaphoreType.DMA,
      pltpu.SemaphoreType.DMA,
      pltpu.SemaphoreType.DMA,
      pltpu.SemaphoreType.DMA,
  ]

  @functools.partial(pl.kernel, out_type=out_type, mesh=mesh,
                     scratch_types=scratch_types,
                     compiler_params=_sc_compiler_params())
  def k(r_hbm, i0_hbm, i1_hbm, i2_hbm, g0, g1, g2,
        iv0, iv1, iv2, b0, b1, b2, sg0, sg1, sw0, sw1):
    wid = lax.axis_index("s") * 2 + lax.axis_index("c")
    base = wid * SPW
    pltpu.sync_copy(i0_hbm.at[pl.ds(base, SPW)], iv0)
    pltpu.sync_copy(i1_hbm.at[pl.ds(base, SPW)], iv1)
    pltpu.sync_copy(i2_hbm.at[pl.ds(base, SPW)], iv2)

    sg = (sg0, sg1)
    sw = (sw0, sw1)

    def fire_gather(c, b):
      off = c * CH
      return (
          pltpu.async_copy(r_hbm.at[iv0.at[pl.ds(off, CH)]], b0.at[b], sg[b]),
          pltpu.async_copy(r_hbm.at[iv1.at[pl.ds(off, CH)]], b1.at[b], sg[b]),
          pltpu.async_copy(r_hbm.at[iv2.at[pl.ds(off, CH)]], b2.at[b], sg[b]),
      )

    def fire_write(c, b):
      off = base + c * CH
      return (
          pltpu.async_copy(b0.at[b], g0.at[pl.ds(off, CH)], sw[b]),
          pltpu.async_copy(b1.at[b], g1.at[pl.ds(off, CH)], sw[b]),
          pltpu.async_copy(b2.at[b], g2.at[pl.ds(off, CH)], sw[b]),
      )

    gcps = {0: fire_gather(0, 0)}
    wcps = {}
    for c in range(NCH):
      b = c % 2
      if c >= 1:
        for cp in wcps.pop(c - 1):
          cp.wait()
      if c + 1 < NCH:
        gcps[c + 1] = fire_gather(c + 1, 1 - b)
      for cp in gcps.pop(c):
        cp.wait()
      wcps[c] = fire_write(c, b)
    for c, cps in wcps.items():
      for cp in cps:
        cp.wait()

  return k(r, i0, i1, i2)


def _arccos(x):
  ax = jnp.abs(x)
  p = jnp.float32(-0.0012624911)
  p = p * ax + jnp.float32(0.0066700901)
  p = p * ax + jnp.float32(-0.0170881256)
  p = p * ax + jnp.float32(0.0308918810)
  p = p * ax + jnp.float32(-0.0501743046)
  p = p * ax + jnp.float32(0.0889789874)
  p = p * ax + jnp.float32(-0.2145988016)
  p = p * ax + jnp.float32(1.5707963050)
  r = jnp.sqrt(jnp.maximum(1.0 - ax, 0.0)) * p
  return jnp.where(x >= 0, r, jnp.float32(np.pi) - r)


def _tc_body(g0_r, g1_r, g2_r, d_r, n1_r, n2_r, w1_r, b1_r, w2_r,
             lo_r, hi_r, c_r, rb_r, acc_r, out_r):
  i = pl.program_id(0)
  x1 = (g0_r[...] + g2_r[...]).astype(jnp.bfloat16)
  x2 = g1_r[...].astype(jnp.bfloat16)
  w1 = w1_r[...]
  h = jnp.dot(x1, w1[:FR, :], preferred_element_type=jnp.float32)
  h = h + jnp.dot(x2, w1[FR:, :], preferred_element_type=jnp.float32)
  h = jnp.tanh(h + b1_r[...])
  m = h * w2_r[...]
  t0m = jnp.sum(m[:, :LH], axis=1, keepdims=True)
  km = jnp.sum(m[:, LH:], axis=1, keepdims=True)

  dot = d_r[...]
  n1 = n1_r[...]
  n2 = n2_r[...]
  cos = dot * lax.rsqrt(n1 * n2)
  theta_l = _arccos(cos * jnp.float32(1.0 / 1.000001))
  theta = jnp.transpose(theta_l)

  t0h = (c_r[0] + t0m) ** 2
  kh = (c_r[1] + km) ** 2
  e = 0.5 * kh * (theta - t0h) ** 2

  rows = lax.broadcasted_iota(jnp.int32, (TB, N_MOL), 0) + rb_r[0] + i * TB
  oh = (rows >= lo_r[...]) & (rows < hi_r[...])
  contrib = jnp.where(oh, e, 0.0)
  part = jnp.sum(contrib, axis=0, keepdims=True)

  @pl.when(i == 0)
  def _():
    out_r[...] = acc_r[...]

  out_r[...] += part


def _tc_compute(g0, g1, g2, d, n1, n2, w1c, b1c, w2r, lo, hi, consts, rb, acc):
  grid = (SL // TB,)
  full = lambda i: (0, 0)
  blk = lambda i: (i, 0)
  lane = lambda i: (0, i)
  return pl.pallas_call(
      _tc_body,
      grid=grid,
      in_specs=[pl.BlockSpec((TB, FR), blk)] * 3
      + [pl.BlockSpec((1, TB), lane)] * 3
      + [
          pl.BlockSpec((2 * FR, 2 * LH), full),
          pl.BlockSpec((1, 2 * LH), full),
          pl.BlockSpec((1, 2 * LH), full),
          pl.BlockSpec((1, N_MOL), full),
          pl.BlockSpec((1, N_MOL), full),
          pl.BlockSpec(memory_space=pltpu.SMEM),
          pl.BlockSpec(memory_space=pltpu.SMEM),
          pl.BlockSpec((1, N_MOL), full),
      ],
      out_specs=pl.BlockSpec((1, N_MOL), full),
      out_shape=jax.ShapeDtypeStruct((1, N_MOL), jnp.float32),
  )(g0, g1, g2, d, n1, n2, w1c, b1c, w2r, lo, hi, consts, rb, acc)


def kernel(r, xyz, t0_W1, t0_b1, t0_W2, t0_b2, k_W1, k_b1, k_W2, k_b2,
           angles, num_angles):
  f32 = jnp.float32
  angles = angles.astype(jnp.int32)
  pad_tri = jnp.array([[0, 1, 2]], dtype=jnp.int32)
  pad = jnp.broadcast_to(pad_tri, (NP - N_ANGLES, 3))
  ap = jnp.concatenate([angles, pad], axis=0)
  i0 = ap[:, 0]
  i1 = ap[:, 1]
  i2 = ap[:, 2]

  xyzt = xyz.astype(f32).T.reshape(3 * N_NODES)
  rf = r.astype(f32)

  w1c = jnp.concatenate([t0_W1, k_W1], axis=1).astype(jnp.bfloat16)
  b1c = jnp.concatenate([t0_b1, k_b1]).reshape(1, 2 * LH)
  w2r = jnp.concatenate([t0_W2[:, 0], k_W2[:, 0]]).reshape(1, 2 * LH)

  ends = jnp.cumsum(num_angles.astype(jnp.int32))
  lo = (ends - num_angles.astype(jnp.int32)).reshape(1, N_MOL)
  hi = ends.reshape(1, N_MOL)

  c0 = np.float32((109.5 * np.pi / 180.0) ** 0.5)
  c1 = np.float32(10.0 ** 0.5)

  d, n1, n2 = _sc_geometry(xyzt, i0, i1, i2)
  d = d.reshape(1, NP)
  n1 = n1.reshape(1, NP)
  n2 = n2.reshape(1, NP)

  consts = jnp.stack([c0 + t0_b2[0], c1 + k_b2[0]]).astype(f32)
  acc = jnp.zeros((1, N_MOL), f32)
  for s in range(NSLICE):
    sl = slice(s * SL, (s + 1) * SL)
    g0, g1, g2 = _sc_gather_slice(rf, i0[sl], i1[sl], i2[sl])
    rb = jnp.array([s * SL], dtype=jnp.int32)
    acc = _tc_compute(g0, g1, g2, d[:, sl], n1[:, sl], n2[:, sl],
                      w1c, b1c, w2r, lo, hi, consts, rb, acc)
  return acc.reshape(N_MOL, 1)


# speedup vs baseline: 8.5842x; 1.2702x over previous
import dataclasses
import functools

import jax
import jax.numpy as jnp
import numpy as np
from jax import lax
from jax.experimental import pallas as pl
from jax.experimental.pallas import tpu as pltpu
from jax.experimental.pallas import tpu_sc as plsc

N_NODES = 8192
FR = 128
LH = 256
N_ANGLES = 130816
N_MOL = 512
NP = 131072
NSLICE = 4
SL = NP // NSLICE
NW = 32
GPW = NP // NW
SPW = SL // NW
CH = 128
NCH = SPW // CH
TB = 2048


def _sc_compiler_params():
  cp = pltpu.CompilerParams()
  if "needs_layout_passes" in pltpu.CompilerParams.__dataclass_fields__:
    cp = dataclasses.replace(cp, needs_layout_passes=False)
  return cp


def _sc_geometry(xyzt, i0, i1, i2):
  mesh = plsc.VectorSubcoreMesh(core_axis_name="c", subcore_axis_name="s")
  f32 = jnp.float32
  out_type = (
      jax.ShapeDtypeStruct((NP,), f32),
      jax.ShapeDtypeStruct((NP,), f32),
      jax.ShapeDtypeStruct((NP,), f32),
  )
  scratch_types = [
      pltpu.VMEM((GPW,), jnp.int32),
      pltpu.VMEM((GPW,), jnp.int32),
      pltpu.VMEM((GPW,), jnp.int32),
      pltpu.VMEM((N_NODES,), f32),
      pltpu.VMEM((N_NODES,), f32),
      pltpu.VMEM((N_NODES,), f32),
      pltpu.VMEM((GPW,), f32),
      pltpu.VMEM((GPW,), f32),
      pltpu.VMEM((GPW,), f32),
  ]

  @functools.partial(pl.kernel, out_type=out_type, mesh=mesh,
                     scratch_types=scratch_types,
                     compiler_params=_sc_compiler_params())
  def k(x_hbm, i0_hbm, i1_hbm, i2_hbm, dh, n1h, n2h,
        iv0, iv1, iv2, xs, ys, zs, dv, n1v, n2v):
    wid = lax.axis_index("s") * 2 + lax.axis_index("c")
    base = wid * GPW
    pltpu.sync_copy(x_hbm.at[pl.ds(0, N_NODES)], xs)
    pltpu.sync_copy(x_hbm.at[pl.ds(N_NODES, N_NODES)], ys)
    pltpu.sync_copy(x_hbm.at[pl.ds(2 * N_NODES, N_NODES)], zs)
    pltpu.sync_copy(i0_hbm.at[pl.ds(base, GPW)], iv0)
    pltpu.sync_copy(i1_hbm.at[pl.ds(base, GPW)], iv1)
    pltpu.sync_copy(i2_hbm.at[pl.ds(base, GPW)], iv2)

    @pl.loop(0, GPW // 16)
    def _(c):
      o = c * 16
      j0 = iv0[pl.ds(o, 16)]
      j1 = iv1[pl.ds(o, 16)]
      j2 = iv2[pl.ds(o, 16)]
      x0 = plsc.load_gather(xs, [j0])
      x1 = plsc.load_gather(xs, [j1])
      x2 = plsc.load_gather(xs, [j2])
      y0 = plsc.load_gather(ys, [j0])
      y1 = plsc.load_gather(ys, [j1])
      y2 = plsc.load_gather(ys, [j2])
      z0 = plsc.load_gather(zs, [j0])
      z1 = plsc.load_gather(zs, [j1])
      z2 = plsc.load_gather(zs, [j2])
      d1x = x1 - x0
      d1y = y1 - y0
      d1z = z1 - z0
      d2x = x2 - x1
      d2y = y2 - y1
      d2z = z2 - z1
      dv[pl.ds(o, 16)] = -(d1x * d2x + d1y * d2y + d1z * d2z)
      n1v[pl.ds(o, 16)] = d1x * d1x + d1y * d1y + d1z * d1z
      n2v[pl.ds(o, 16)] = d2x * d2x + d2y * d2y + d2z * d2z

    pltpu.sync_copy(dv, dh.at[pl.ds(base, GPW)])
    pltpu.sync_copy(n1v, n1h.at[pl.ds(base, GPW)])
    pltpu.sync_copy(n2v, n2h.at[pl.ds(base, GPW)])

  return k(xyzt, i0, i1, i2)


def _sc_gather_slice(r, i0, i1, i2):
  mesh = plsc.VectorSubcoreMesh(core_axis_name="c", subcore_axis_name="s")
  f32 = jnp.float32
  out_type = (
      jax.ShapeDtypeStruct((SL, FR), f32),
      jax.ShapeDtypeStruct((SL, FR), f32),
      jax.ShapeDtypeStruct((SL, FR), f32),
  )
  scratch_types = [
      pltpu.VMEM((SPW,), jnp.int32),
      pltpu.VMEM((SPW,), jnp.int32),
      pltpu.VMEM((SPW,), jnp.int32),
      pltpu.VMEM((2, CH, FR), f32),
      pltpu.VMEM((2, CH, FR), f32),
      pltpu.VMEM((2, CH, FR), f32),
      pltpu.SemaphoreType.DMA,
      pltpu.SemaphoreType.DMA,
      pltpu.SemaphoreType.DMA,
      pltpu.SemaphoreType.DMA,
  ]

  @functools.partial(pl.kernel, out_type=out_type, mesh=mesh,
                     scratch_types=scratch_types,
                     compiler_params=_sc_compiler_params())
  def k(r_hbm, i0_hbm, i1_hbm, i2_hbm, g0, g1, g2,
        iv0, iv1, iv2, b0, b1, b2, sg0, sg1, sw0, sw1):
    wid = lax.axis_index("s") * 2 + lax.axis_index("c")
    base = wid * SPW
    pltpu.sync_copy(i0_hbm.at[pl.ds(base, SPW)], iv0)
    pltpu.sync_copy(i1_hbm.at[pl.ds(base, SPW)], iv1)
    pltpu.sync_copy(i2_hbm.at[pl.ds(base, SPW)], iv2)

    sg = (sg0, sg1)
    sw = (sw0, sw1)

    def fire_gather(c, b):
      off = c * CH
      return (
          pltpu.async_copy(r_hbm.at[iv0.at[pl.ds(off, CH)]], b0.at[b], sg[b]),
          pltpu.async_copy(r_hbm.at[iv1.at[pl.ds(off, CH)]], b1.at[b], sg[b]),
          pltpu.async_copy(r_hbm.at[iv2.at[pl.ds(off, CH)]], b2.at[b], sg[b]),
      )

    def fire_write(c, b):
      off = base + c * CH
      return (
          pltpu.async_copy(b0.at[b], g0.at[pl.ds(off, CH)], sw[b]),
          pltpu.async_copy(b1.at[b], g1.at[pl.ds(off, CH)], sw[b]),
          pltpu.async_copy(b2.at[b], g2.at[pl.ds(off, CH)], sw[b]),
      )

    gcps = {0: fire_gather(0, 0)}
    wcps = {}
    for c in range(NCH):
      b = c % 2
      if c >= 1:
        for cp in wcps.pop(c - 1):
          cp.wait()
      if c + 1 < NCH:
        gcps[c + 1] = fire_gather(c + 1, 1 - b)
      for cp in gcps.pop(c):
        cp.wait()
      wcps[c] = fire_write(c, b)
    for c, cps in wcps.items():
      for cp in cps:
        cp.wait()

  return k(r, i0, i1, i2)


def _arccos(x):
  ax = jnp.abs(x)
  p = jnp.float32(-0.0012624911)
  p = p * ax + jnp.float32(0.0066700901)
  p = p * ax + jnp.float32(-0.0170881256)
  p = p * ax + jnp.float32(0.0308918810)
  p = p * ax + jnp.float32(-0.0501743046)
  p = p * ax + jnp.float32(0.0889789874)
  p = p * ax + jnp.float32(-0.2145988016)
  p = p * ax + jnp.float32(1.5707963050)
  r = jnp.sqrt(jnp.maximum(1.0 - ax, 0.0)) * p
  return jnp.where(x >= 0, r, jnp.float32(np.pi) - r)


WIN = 256
NMP = N_MOL + 128


def _tc_body(g0_r, g1_r, g2_r, d_r, n1_r, n2_r, w1_r, b1_r, w2_r,
             lo_r, hi_r, c_r, rb_r, wb_r, acc_r, out_r):
  i = pl.program_id(0)
  x1 = (g0_r[...] + g2_r[...]).astype(jnp.bfloat16)
  x2 = g1_r[...].astype(jnp.bfloat16)
  w1 = w1_r[...]
  h = jnp.dot(x1, w1[:FR, :], preferred_element_type=jnp.float32)
  h = h + jnp.dot(x2, w1[FR:, :], preferred_element_type=jnp.float32)
  h = jnp.tanh(h + b1_r[...])
  m = h * w2_r[...]
  t0m = jnp.sum(m[:, :LH], axis=1, keepdims=True)
  km = jnp.sum(m[:, LH:], axis=1, keepdims=True)
  t0l = jnp.transpose(t0m)
  kml = jnp.transpose(km)

  dot = d_r[...]
  n1 = n1_r[...]
  n2 = n2_r[...]
  cos = dot * lax.rsqrt(n1 * n2)
  theta = _arccos(cos * jnp.float32(1.0 / 1.000001))
  t0h = (c_r[0] + t0l) ** 2
  kh = (c_r[1] + kml) ** 2
  e = 0.5 * kh * (theta - t0h) ** 2

  wb = pl.multiple_of(wb_r[i], 128)
  low = lo_r[0:1, pl.ds(wb, WIN)]
  high = hi_r[0:1, pl.ds(wb, WIN)]
  rows = lax.broadcasted_iota(jnp.int32, (TB, WIN), 0) + rb_r[0] + i * TB
  oh = (rows >= low) & (rows < high)
  maskf = jnp.where(oh, jnp.float32(1.0), jnp.float32(0.0))
  part = lax.dot_general(e, maskf, (((1,), (0,)), ((), ())),
                         preferred_element_type=jnp.float32)

  @pl.when(i == 0)
  def _():
    out_r[...] = acc_r[...]

  out_r[0:1, pl.ds(wb, WIN)] += part


def _tc_compute(g0, g1, g2, d, n1, n2, w1c, b1c, w2r, lo, hi, consts, rb, wb,
                acc):
  grid = (SL // TB,)
  full = lambda i: (0, 0)
  blk = lambda i: (i, 0)
  lane = lambda i: (0, i)
  return pl.pallas_call(
      _tc_body,
      grid=grid,
      in_specs=[pl.BlockSpec((TB, FR), blk)] * 3
      + [pl.BlockSpec((1, TB), lane)] * 3
      + [
          pl.BlockSpec((2 * FR, 2 * LH), full),
          pl.BlockSpec((1, 2 * LH), full),
          pl.BlockSpec((1, 2 * LH), full),
          pl.BlockSpec((1, NMP), full),
          pl.BlockSpec((1, NMP), full),
          pl.BlockSpec(memory_space=pltpu.SMEM),
          pl.BlockSpec(memory_space=pltpu.SMEM),
          pl.BlockSpec(memory_space=pltpu.SMEM),
          pl.BlockSpec((1, NMP), full),
      ],
      out_specs=pl.BlockSpec((1, NMP), full),
      out_shape=jax.ShapeDtypeStruct((1, NMP), jnp.float32),
  )(g0, g1, g2, d, n1, n2, w1c, b1c, w2r, lo, hi, consts, rb, wb, acc)


def kernel(r, xyz, t0_W1, t0_b1, t0_W2, t0_b2, k_W1, k_b1, k_W2, k_b2,
           angles, num_angles):
  f32 = jnp.float32
  angles = angles.astype(jnp.int32)
  pad_tri = jnp.array([[0, 1, 2]], dtype=jnp.int32)
  pad = jnp.broadcast_to(pad_tri, (NP - N_ANGLES, 3))
  ap = jnp.concatenate([angles, pad], axis=0)
  i0 = ap[:, 0]
  i1 = ap[:, 1]
  i2 = ap[:, 2]

  xyzt = xyz.astype(f32).T.reshape(3 * N_NODES)
  rf = r.astype(f32)

  w1c = jnp.concatenate([t0_W1, k_W1], axis=1).astype(jnp.bfloat16)
  b1c = jnp.concatenate([t0_b1, k_b1]).reshape(1, 2 * LH)
  w2r = jnp.concatenate([t0_W2[:, 0], k_W2[:, 0]]).reshape(1, 2 * LH)

  ends = jnp.cumsum(num_angles.astype(jnp.int32))
  starts = ends - num_angles.astype(jnp.int32)
  lo = jnp.pad(starts, (0, NMP - N_MOL),
               constant_values=np.int32(NP + 1)).reshape(1, NMP)
  hi = jnp.pad(ends, (0, NMP - N_MOL),
               constant_values=np.int32(0)).reshape(1, NMP)
  blk_rows = jnp.arange(NP // TB, dtype=jnp.int32) * TB
  base_mol = jnp.searchsorted(ends, blk_rows, side="right").astype(jnp.int32)
  wb_all = jnp.minimum(base_mol // 128 * 128, np.int32(N_MOL - 128))

  c0 = np.float32((109.5 * np.pi / 180.0) ** 0.5)
  c1 = np.float32(10.0 ** 0.5)

  d, n1, n2 = _sc_geometry(xyzt, i0, i1, i2)
  d = d.reshape(1, NP)
  n1 = n1.reshape(1, NP)
  n2 = n2.reshape(1, NP)

  consts = jnp.stack([c0 + t0_b2[0], c1 + k_b2[0]]).astype(f32)
  nblk = SL // TB
  acc = jnp.zeros((1, NMP), f32)
  for s in range(NSLICE):
    sl = slice(s * SL, (s + 1) * SL)
    g0, g1, g2 = _sc_gather_slice(rf, i0[sl], i1[sl], i2[sl])
    rb = jnp.array([s * SL], dtype=jnp.int32)
    wb = wb_all[s * nblk:(s + 1) * nblk]
    acc = _tc_compute(g0, g1, g2, d[:, sl], n1[:, sl], n2[:, sl],
                      w1c, b1c, w2r, lo, hi, consts, rb, wb, acc)
  return acc[:, :N_MOL].reshape(N_MOL, 1)


# speedup vs baseline: 8.9260x; 1.0398x over previous
import dataclasses
import functools

import jax
import jax.numpy as jnp
import numpy as np
from jax import lax
from jax.experimental import pallas as pl
from jax.experimental.pallas import tpu as pltpu
from jax.experimental.pallas import tpu_sc as plsc

N_NODES = 8192
FR = 128
LH = 256
N_ANGLES = 130816
N_MOL = 512
NP = 131072
NSLICE = 4
SL = NP // NSLICE
NW = 32
GPW = NP // NW
SPW = SL // NW
CH = 128
NCH = SPW // CH
TB = 2048


def _sc_compiler_params():
  cp = pltpu.CompilerParams()
  if "needs_layout_passes" in pltpu.CompilerParams.__dataclass_fields__:
    cp = dataclasses.replace(cp, needs_layout_passes=False)
  return cp


def _sc_gather_slice(r, xyzt, i0, i1, i2):
  mesh = plsc.VectorSubcoreMesh(core_axis_name="c", subcore_axis_name="s")
  f32 = jnp.float32
  out_type = (
      jax.ShapeDtypeStruct((SL, FR), f32),
      jax.ShapeDtypeStruct((SL, FR), f32),
      jax.ShapeDtypeStruct((SL, FR), f32),
      jax.ShapeDtypeStruct((SL,), f32),
      jax.ShapeDtypeStruct((SL,), f32),
      jax.ShapeDtypeStruct((SL,), f32),
  )
  scratch_types = [
      pltpu.VMEM((SPW,), jnp.int32),
      pltpu.VMEM((SPW,), jnp.int32),
      pltpu.VMEM((SPW,), jnp.int32),
      pltpu.VMEM((2, CH, FR), f32),
      pltpu.VMEM((2, CH, FR), f32),
      pltpu.VMEM((2, CH, FR), f32),
      pltpu.VMEM((N_NODES,), f32),
      pltpu.VMEM((N_NODES,), f32),
      pltpu.VMEM((N_NODES,), f32),
      pltpu.VMEM((SPW,), f32),
      pltpu.VMEM((SPW,), f32),
      pltpu.VMEM((SPW,), f32),
      pltpu.SemaphoreType.DMA,
      pltpu.SemaphoreType.DMA,
      pltpu.SemaphoreType.DMA,
      pltpu.SemaphoreType.DMA,
  ]

  @functools.partial(pl.kernel, out_type=out_type, mesh=mesh,
                     scratch_types=scratch_types,
                     compiler_params=_sc_compiler_params())
  def k(r_hbm, x_hbm, i0_hbm, i1_hbm, i2_hbm, g0, g1, g2, dh, n1h, n2h,
        iv0, iv1, iv2, b0, b1, b2, xs, ys, zs, dv, n1v, n2v,
        sg0, sg1, sw0, sw1):
    wid = lax.axis_index("s") * 2 + lax.axis_index("c")
    base = wid * SPW
    pltpu.sync_copy(i0_hbm.at[pl.ds(base, SPW)], iv0)
    pltpu.sync_copy(i1_hbm.at[pl.ds(base, SPW)], iv1)
    pltpu.sync_copy(i2_hbm.at[pl.ds(base, SPW)], iv2)
    pltpu.sync_copy(x_hbm.at[pl.ds(0, N_NODES)], xs)
    pltpu.sync_copy(x_hbm.at[pl.ds(N_NODES, N_NODES)], ys)
    pltpu.sync_copy(x_hbm.at[pl.ds(2 * N_NODES, N_NODES)], zs)

    def geometry(c):
      for kk in range(CH // 16):
        o = c * CH + 16 * kk
        j0 = iv0[pl.ds(o, 16)]
        j1 = iv1[pl.ds(o, 16)]
        j2 = iv2[pl.ds(o, 16)]
        x0 = plsc.load_gather(xs, [j0])
        x1 = plsc.load_gather(xs, [j1])
        x2 = plsc.load_gather(xs, [j2])
        y0 = plsc.load_gather(ys, [j0])
        y1 = plsc.load_gather(ys, [j1])
        y2 = plsc.load_gather(ys, [j2])
        z0 = plsc.load_gather(zs, [j0])
        z1 = plsc.load_gather(zs, [j1])
        z2 = plsc.load_gather(zs, [j2])
        d1x = x1 - x0
        d1y = y1 - y0
        d1z = z1 - z0
        d2x = x2 - x1
        d2y = y2 - y1
        d2z = z2 - z1
        dv[pl.ds(o, 16)] = -(d1x * d2x + d1y * d2y + d1z * d2z)
        n1v[pl.ds(o, 16)] = d1x * d1x + d1y * d1y + d1z * d1z
        n2v[pl.ds(o, 16)] = d2x * d2x + d2y * d2y + d2z * d2z

    sg = (sg0, sg1)
    sw = (sw0, sw1)

    def fire_gather(c, b):
      off = c * CH
      return (
          pltpu.async_copy(r_hbm.at[iv0.at[pl.ds(off, CH)]], b0.at[b], sg[b]),
          pltpu.async_copy(r_hbm.at[iv1.at[pl.ds(off, CH)]], b1.at[b], sg[b]),
          pltpu.async_copy(r_hbm.at[iv2.at[pl.ds(off, CH)]], b2.at[b], sg[b]),
      )

    def fire_write(c, b):
      off = base + c * CH
      return (
          pltpu.async_copy(b0.at[b], g0.at[pl.ds(off, CH)], sw[b]),
          pltpu.async_copy(b1.at[b], g1.at[pl.ds(off, CH)], sw[b]),
          pltpu.async_copy(b2.at[b], g2.at[pl.ds(off, CH)], sw[b]),
      )

    gcps = {0: fire_gather(0, 0)}
    wcps = {}
    for c in range(NCH):
      b = c % 2
      if c >= 1:
        for cp in wcps.pop(c - 1):
          cp.wait()
      if c + 1 < NCH:
        gcps[c + 1] = fire_gather(c + 1, 1 - b)
      geometry(c)
      for cp in gcps.pop(c):
        cp.wait()
      wcps[c] = fire_write(c, b)
    for c, cps in wcps.items():
      for cp in cps:
        cp.wait()

    pltpu.sync_copy(dv, dh.at[pl.ds(base, SPW)])
    pltpu.sync_copy(n1v, n1h.at[pl.ds(base, SPW)])
    pltpu.sync_copy(n2v, n2h.at[pl.ds(base, SPW)])

  return k(r, xyzt, i0, i1, i2)


def _arccos(x):
  ax = jnp.abs(x)
  p = jnp.float32(-0.0012624911)
  p = p * ax + jnp.float32(0.0066700901)
  p = p * ax + jnp.float32(-0.0170881256)
  p = p * ax + jnp.float32(0.0308918810)
  p = p * ax + jnp.float32(-0.0501743046)
  p = p * ax + jnp.float32(0.0889789874)
  p = p * ax + jnp.float32(-0.2145988016)
  p = p * ax + jnp.float32(1.5707963050)
  r = jnp.sqrt(jnp.maximum(1.0 - ax, 0.0)) * p
  return jnp.where(x >= 0, r, jnp.float32(np.pi) - r)


WIN = 256
NMP = N_MOL + 128


def _tc_body(g0_r, g1_r, g2_r, d_r, n1_r, n2_r, w1_r, b1_r, w2_r,
             lo_r, hi_r, c_r, rb_r, wb_r, acc_r, out_r):
  i = pl.program_id(0)
  x1 = (g0_r[...] + g2_r[...]).astype(jnp.bfloat16)
  x2 = g1_r[...].astype(jnp.bfloat16)
  x = jnp.concatenate([x1, x2], axis=1)
  h = jnp.dot(x, w1_r[...], preferred_element_type=jnp.float32)
  h = jnp.tanh(h + b1_r[...])
  m = h * w2_r[...]
  t0m = jnp.sum(m[:, :LH], axis=1, keepdims=True)
  km = jnp.sum(m[:, LH:], axis=1, keepdims=True)
  t0l = jnp.transpose(t0m)
  kml = jnp.transpose(km)

  dot = d_r[...]
  n1 = n1_r[...]
  n2 = n2_r[...]
  cos = dot * lax.rsqrt(n1 * n2)
  theta = _arccos(cos * jnp.float32(1.0 / 1.000001))
  t0h = (c_r[0] + t0l) ** 2
  kh = (c_r[1] + kml) ** 2
  e = 0.5 * kh * (theta - t0h) ** 2

  wb = pl.multiple_of(wb_r[i], 128)
  shift = rb_r[0] + i * TB
  low = lo_r[0:1, pl.ds(wb, WIN)] - shift
  high = hi_r[0:1, pl.ds(wb, WIN)] - shift
  rows = lax.broadcasted_iota(jnp.int32, (TB, WIN), 0)
  oh = (rows >= low) & (rows < high)
  maskf = jnp.where(oh, jnp.float32(1.0), jnp.float32(0.0))
  part = lax.dot_general(e, maskf, (((1,), (0,)), ((), ())),
                         preferred_element_type=jnp.float32)

  @pl.when(i == 0)
  def _():
    out_r[...] = acc_r[...]

  out_r[0:1, pl.ds(wb, WIN)] += part


def _tc_compute(g0, g1, g2, d, n1, n2, w1c, b1c, w2r, lo, hi, consts, rb, wb,
                acc):
  grid = (SL // TB,)
  full = lambda i: (0, 0)
  blk = lambda i: (i, 0)
  lane = lambda i: (0, i)
  return pl.pallas_call(
      _tc_body,
      grid=grid,
      in_specs=[pl.BlockSpec((TB, FR), blk)] * 3
      + [pl.BlockSpec((1, TB), lane)] * 3
      + [
          pl.BlockSpec((2 * FR, 2 * LH), full),
          pl.BlockSpec((1, 2 * LH), full),
          pl.BlockSpec((1, 2 * LH), full),
          pl.BlockSpec((1, NMP), full),
          pl.BlockSpec((1, NMP), full),
          pl.BlockSpec(memory_space=pltpu.SMEM),
          pl.BlockSpec(memory_space=pltpu.SMEM),
          pl.BlockSpec(memory_space=pltpu.SMEM),
          pl.BlockSpec((1, NMP), full),
      ],
      out_specs=pl.BlockSpec((1, NMP), full),
      out_shape=jax.ShapeDtypeStruct((1, NMP), jnp.float32),
  )(g0, g1, g2, d, n1, n2, w1c, b1c, w2r, lo, hi, consts, rb, wb, acc)


def kernel(r, xyz, t0_W1, t0_b1, t0_W2, t0_b2, k_W1, k_b1, k_W2, k_b2,
           angles, num_angles):
  f32 = jnp.float32
  angles = angles.astype(jnp.int32)
  pad_tri = jnp.array([[0, 1, 2]], dtype=jnp.int32)
  pad = jnp.broadcast_to(pad_tri, (NP - N_ANGLES, 3))
  ap = jnp.concatenate([angles, pad], axis=0)
  i0 = ap[:, 0]
  i1 = ap[:, 1]
  i2 = ap[:, 2]

  xyzt = xyz.astype(f32).T.reshape(3 * N_NODES)
  rf = r.astype(f32)

  w1c = jnp.concatenate([t0_W1, k_W1], axis=1).astype(jnp.bfloat16)
  b1c = jnp.concatenate([t0_b1, k_b1]).reshape(1, 2 * LH)
  w2r = jnp.concatenate([t0_W2[:, 0], k_W2[:, 0]]).reshape(1, 2 * LH)

  ends = jnp.cumsum(num_angles.astype(jnp.int32))
  starts = ends - num_angles.astype(jnp.int32)
  lo = jnp.pad(starts, (0, NMP - N_MOL),
               constant_values=np.int32(NP + 1)).reshape(1, NMP)
  hi = jnp.pad(ends, (0, NMP - N_MOL),
               constant_values=np.int32(0)).reshape(1, NMP)
  blk_rows = jnp.arange(NP // TB, dtype=jnp.int32) * TB
  base_mol = jnp.searchsorted(ends, blk_rows, side="right").astype(jnp.int32)
  wb_all = jnp.minimum(base_mol // 128 * 128, np.int32(N_MOL - 128))

  c0 = np.float32((109.5 * np.pi / 180.0) ** 0.5)
  c1 = np.float32(10.0 ** 0.5)

  consts = jnp.stack([c0 + t0_b2[0], c1 + k_b2[0]]).astype(f32)
  nblk = SL // TB
  acc = jnp.zeros((1, NMP), f32)
  for s in range(NSLICE):
    sl = slice(s * SL, (s + 1) * SL)
    g0, g1, g2, d, n1, n2 = _sc_gather_slice(rf, xyzt,
                                             i0[sl], i1[sl], i2[sl])
    rb = jnp.array([s * SL], dtype=jnp.int32)
    wb = wb_all[s * nblk:(s + 1) * nblk]
    acc = _tc_compute(g0, g1, g2, d.reshape(1, SL), n1.reshape(1, SL),
                      n2.reshape(1, SL), w1c, b1c, w2r, lo, hi, consts,
                      rb, wb, acc)
  return acc[:, :N_MOL].reshape(N_MOL, 1)


# speedup vs baseline: 9.0667x; 1.0158x over previous
import dataclasses
import functools

import jax
import jax.numpy as jnp
import numpy as np
from jax import lax
from jax.experimental import pallas as pl
from jax.experimental.pallas import tpu as pltpu
from jax.experimental.pallas import tpu_sc as plsc

N_NODES = 8192
FR = 128
LH = 256
N_ANGLES = 130816
N_MOL = 512
NP = 131072
NSLICE = 4
SL = NP // NSLICE
NW = 32
GPW = NP // NW
SPW = SL // NW
CH = 64
NB = 3
NCH = SPW // CH
TB = 2048


def _sc_compiler_params():
  cp = pltpu.CompilerParams()
  if "needs_layout_passes" in pltpu.CompilerParams.__dataclass_fields__:
    cp = dataclasses.replace(cp, needs_layout_passes=False)
  return cp


def _sc_gather_slice(r, xyzt, i0, i1, i2):
  mesh = plsc.VectorSubcoreMesh(core_axis_name="c", subcore_axis_name="s")
  f32 = jnp.float32
  out_type = (
      jax.ShapeDtypeStruct((SL, FR), f32),
      jax.ShapeDtypeStruct((SL, FR), f32),
      jax.ShapeDtypeStruct((SL, FR), f32),
      jax.ShapeDtypeStruct((SL,), f32),
      jax.ShapeDtypeStruct((SL,), f32),
      jax.ShapeDtypeStruct((SL,), f32),
  )
  scratch_types = [
      pltpu.VMEM((SPW,), jnp.int32),
      pltpu.VMEM((SPW,), jnp.int32),
      pltpu.VMEM((SPW,), jnp.int32),
      pltpu.VMEM((NB, CH, FR), f32),
      pltpu.VMEM((NB, CH, FR), f32),
      pltpu.VMEM((NB, CH, FR), f32),
      pltpu.VMEM((N_NODES,), f32),
      pltpu.VMEM((N_NODES,), f32),
      pltpu.VMEM((N_NODES,), f32),
      pltpu.VMEM((SPW,), f32),
      pltpu.VMEM((SPW,), f32),
      pltpu.VMEM((SPW,), f32),
      pltpu.SemaphoreType.DMA,
      pltpu.SemaphoreType.DMA,
      pltpu.SemaphoreType.DMA,
      pltpu.SemaphoreType.DMA,
      pltpu.SemaphoreType.DMA,
      pltpu.SemaphoreType.DMA,
      pltpu.SemaphoreType.DMA,
  ]

  @functools.partial(pl.kernel, out_type=out_type, mesh=mesh,
                     scratch_types=scratch_types,
                     compiler_params=_sc_compiler_params())
  def k(r_hbm, x_hbm, i0_hbm, i1_hbm, i2_hbm, g0, g1, g2, dh, n1h, n2h,
        iv0, iv1, iv2, b0, b1, b2, xs, ys, zs, dv, n1v, n2v,
        sg0, sg1, sg2, sw0, sw1, sw2, sx):
    wid = lax.axis_index("s") * 2 + lax.axis_index("c")
    base = wid * SPW
    pltpu.sync_copy(i0_hbm.at[pl.ds(base, SPW)], iv0)
    pltpu.sync_copy(i1_hbm.at[pl.ds(base, SPW)], iv1)
    pltpu.sync_copy(i2_hbm.at[pl.ds(base, SPW)], iv2)
    xcps = (
        pltpu.async_copy(x_hbm.at[pl.ds(0, N_NODES)], xs, sx),
        pltpu.async_copy(x_hbm.at[pl.ds(N_NODES, N_NODES)], ys, sx),
        pltpu.async_copy(x_hbm.at[pl.ds(2 * N_NODES, N_NODES)], zs, sx),
    )

    def geometry(c):
      for kk in range(CH // 16):
        o = c * CH + 16 * kk
        j0 = iv0[pl.ds(o, 16)]
        j1 = iv1[pl.ds(o, 16)]
        j2 = iv2[pl.ds(o, 16)]
        x0 = plsc.load_gather(xs, [j0])
        x1 = plsc.load_gather(xs, [j1])
        x2 = plsc.load_gather(xs, [j2])
        y0 = plsc.load_gather(ys, [j0])
        y1 = plsc.load_gather(ys, [j1])
        y2 = plsc.load_gather(ys, [j2])
        z0 = plsc.load_gather(zs, [j0])
        z1 = plsc.load_gather(zs, [j1])
        z2 = plsc.load_gather(zs, [j2])
        d1x = x1 - x0
        d1y = y1 - y0
        d1z = z1 - z0
        d2x = x2 - x1
        d2y = y2 - y1
        d2z = z2 - z1
        dv[pl.ds(o, 16)] = -(d1x * d2x + d1y * d2y + d1z * d2z)
        n1v[pl.ds(o, 16)] = d1x * d1x + d1y * d1y + d1z * d1z
        n2v[pl.ds(o, 16)] = d2x * d2x + d2y * d2y + d2z * d2z

    sg = (sg0, sg1, sg2)
    sw = (sw0, sw1, sw2)

    def fire_gather(c, b):
      off = c * CH
      return (
          pltpu.async_copy(r_hbm.at[iv0.at[pl.ds(off, CH)]], b0.at[b], sg[b]),
          pltpu.async_copy(r_hbm.at[iv1.at[pl.ds(off, CH)]], b1.at[b], sg[b]),
          pltpu.async_copy(r_hbm.at[iv2.at[pl.ds(off, CH)]], b2.at[b], sg[b]),
      )

    def fire_write(c, b):
      off = base + c * CH
      return (
          pltpu.async_copy(b0.at[b], g0.at[pl.ds(off, CH)], sw[b]),
          pltpu.async_copy(b1.at[b], g1.at[pl.ds(off, CH)], sw[b]),
          pltpu.async_copy(b2.at[b], g2.at[pl.ds(off, CH)], sw[b]),
      )

    gcps = {0: fire_gather(0, 0), 1: fire_gather(1, 1)}
    wcps = {}
    for c in range(NCH):
      b = c % NB
      if c + 2 < NCH:
        if c >= 1:
          for cp in wcps.pop(c - 1):
            cp.wait()
        gcps[c + 2] = fire_gather(c + 2, (c + 2) % NB)
      if c == 0:
        for cp in xcps:
          cp.wait()
      geometry(c)
      for cp in gcps.pop(c):
        cp.wait()
      wcps[c] = fire_write(c, b)
    for c, cps in wcps.items():
      for cp in cps:
        cp.wait()

    pltpu.sync_copy(dv, dh.at[pl.ds(base, SPW)])
    pltpu.sync_copy(n1v, n1h.at[pl.ds(base, SPW)])
    pltpu.sync_copy(n2v, n2h.at[pl.ds(base, SPW)])

  return k(r, xyzt, i0, i1, i2)


def _arccos(x):
  ax = jnp.abs(x)
  p = jnp.float32(-0.0012624911)
  p = p * ax + jnp.float32(0.0066700901)
  p = p * ax + jnp.float32(-0.0170881256)
  p = p * ax + jnp.float32(0.0308918810)
  p = p * ax + jnp.float32(-0.0501743046)
  p = p * ax + jnp.float32(0.0889789874)
  p = p * ax + jnp.float32(-0.2145988016)
  p = p * ax + jnp.float32(1.5707963050)
  r = jnp.sqrt(jnp.maximum(1.0 - ax, 0.0)) * p
  return jnp.where(x >= 0, r, jnp.float32(np.pi) - r)


WIN = 256
NMP = N_MOL + 128


def _tc_body(g0_r, g1_r, g2_r, d_r, n1_r, n2_r, w1_r, b1_r, w2_r,
             lo_r, hi_r, c_r, rb_r, wb_r, acc_r, out_r):
  i = pl.program_id(0)
  x1 = (g0_r[...] + g2_r[...]).astype(jnp.bfloat16)
  x2 = g1_r[...].astype(jnp.bfloat16)
  x = jnp.concatenate([x1, x2], axis=1)
  h = jnp.dot(x, w1_r[...], preferred_element_type=jnp.float32)
  h = jnp.tanh(h + b1_r[...])
  m = h * w2_r[...]
  t0m = jnp.sum(m[:, :LH], axis=1, keepdims=True)
  km = jnp.sum(m[:, LH:], axis=1, keepdims=True)
  t0l = jnp.transpose(t0m)
  kml = jnp.transpose(km)

  dot = d_r[...]
  n1 = n1_r[...]
  n2 = n2_r[...]
  cos = dot * lax.rsqrt(n1 * n2)
  theta = _arccos(cos * jnp.float32(1.0 / 1.000001))
  t0h = (c_r[0] + t0l) ** 2
  kh = (c_r[1] + kml) ** 2
  e = 0.5 * kh * (theta - t0h) ** 2

  wb = pl.multiple_of(wb_r[i], 128)
  shift = rb_r[0] + i * TB
  low = lo_r[0:1, pl.ds(wb, WIN)] - shift
  high = hi_r[0:1, pl.ds(wb, WIN)] - shift
  rows = lax.broadcasted_iota(jnp.int32, (TB, WIN), 0)
  oh = (rows >= low) & (rows < high)
  maskf = jnp.where(oh, jnp.float32(1.0), jnp.float32(0.0))
  part = lax.dot_general(e, maskf, (((1,), (0,)), ((), ())),
                         preferred_element_type=jnp.float32)

  @pl.when(i == 0)
  def _():
    out_r[...] = acc_r[...]

  out_r[0:1, pl.ds(wb, WIN)] += part


def _tc_compute(g0, g1, g2, d, n1, n2, w1c, b1c, w2r, lo, hi, consts, rb, wb,
                acc):
  grid = (SL // TB,)
  full = lambda i: (0, 0)
  blk = lambda i: (i, 0)
  lane = lambda i: (0, i)
  return pl.pallas_call(
      _tc_body,
      grid=grid,
      in_specs=[pl.BlockSpec((TB, FR), blk)] * 3
      + [pl.BlockSpec((1, TB), lane)] * 3
      + [
          pl.BlockSpec((2 * FR, 2 * LH), full),
          pl.BlockSpec((1, 2 * LH), full),
          pl.BlockSpec((1, 2 * LH), full),
          pl.BlockSpec((1, NMP), full),
          pl.BlockSpec((1, NMP), full),
          pl.BlockSpec(memory_space=pltpu.SMEM),
          pl.BlockSpec(memory_space=pltpu.SMEM),
          pl.BlockSpec(memory_space=pltpu.SMEM),
          pl.BlockSpec((1, NMP), full),
      ],
      out_specs=pl.BlockSpec((1, NMP), full),
      out_shape=jax.ShapeDtypeStruct((1, NMP), jnp.float32),
  )(g0, g1, g2, d, n1, n2, w1c, b1c, w2r, lo, hi, consts, rb, wb, acc)


def kernel(r, xyz, t0_W1, t0_b1, t0_W2, t0_b2, k_W1, k_b1, k_W2, k_b2,
           angles, num_angles):
  f32 = jnp.float32
  angles = angles.astype(jnp.int32)
  pad_tri = jnp.array([[0, 1, 2]], dtype=jnp.int32)
  pad = jnp.broadcast_to(pad_tri, (NP - N_ANGLES, 3))
  ap = jnp.concatenate([angles, pad], axis=0)
  i0 = ap[:, 0]
  i1 = ap[:, 1]
  i2 = ap[:, 2]

  xyzt = xyz.astype(f32).T.reshape(3 * N_NODES)
  rf = r.astype(f32)

  w1c = jnp.concatenate([t0_W1, k_W1], axis=1).astype(jnp.bfloat16)
  b1c = jnp.concatenate([t0_b1, k_b1]).reshape(1, 2 * LH)
  w2r = jnp.concatenate([t0_W2[:, 0], k_W2[:, 0]]).reshape(1, 2 * LH)

  ends = jnp.cumsum(num_angles.astype(jnp.int32))
  starts = ends - num_angles.astype(jnp.int32)
  lo = jnp.pad(starts, (0, NMP - N_MOL),
               constant_values=np.int32(NP + 1)).reshape(1, NMP)
  hi = jnp.pad(ends, (0, NMP - N_MOL),
               constant_values=np.int32(0)).reshape(1, NMP)
  blk_rows = jnp.arange(NP // TB, dtype=jnp.int32) * TB
  base_mol = jnp.searchsorted(ends, blk_rows, side="right").astype(jnp.int32)
  wb_all = jnp.minimum(base_mol // 128 * 128, np.int32(N_MOL - 128))

  c0 = np.float32((109.5 * np.pi / 180.0) ** 0.5)
  c1 = np.float32(10.0 ** 0.5)

  consts = jnp.stack([c0 + t0_b2[0], c1 + k_b2[0]]).astype(f32)
  nblk = SL // TB
  acc = jnp.zeros((1, NMP), f32)
  for s in range(NSLICE):
    sl = slice(s * SL, (s + 1) * SL)
    g0, g1, g2, d, n1, n2 = _sc_gather_slice(rf, xyzt,
                                             i0[sl], i1[sl], i2[sl])
    rb = jnp.array([s * SL], dtype=jnp.int32)
    wb = wb_all[s * nblk:(s + 1) * nblk]
    acc = _tc_compute(g0, g1, g2, d.reshape(1, SL), n1.reshape(1, SL),
                      n2.reshape(1, SL), w1c, b1c, w2r, lo, hi, consts,
                      rb, wb, acc)
  return acc[:, :N_MOL].reshape(N_MOL, 1)
